# baseline (speedup 1.0000x reference)
import jax
import jax.numpy as jnp
from jax.experimental import pallas as pl
from jax.experimental.pallas import tpu as pltpu

_SIGMA = 0.1
_NCORES = 2
_NSPLIT = 2


def _round_up(v, m):
    return ((v + m - 1) // m) * m


def _noise_linear_kernel(x_ref, w_ref, nw_ref, b_ref, nb_ref, o_ref):
    weff = (w_ref[...] + _SIGMA * nw_ref[...]).astype(jnp.bfloat16)
    beff = b_ref[...] + _SIGMA * nb_ref[...]
    o_ref[...] = (
        jnp.dot(x_ref[...].astype(jnp.bfloat16), weff,
                preferred_element_type=jnp.float32)
        + beff
    )


def kernel(x, w_t, bias2d, noise_w_t, noise_b2d):
    B, K = x.shape
    Kw, N = w_t.shape
    assert K == Kw
    assert N % _NSPLIT == 0
    nh = N // _NSPLIT

    bt = _round_up(B, 8 * _NCORES) // _NCORES
    Bp = bt * _NCORES
    x_p = x if Bp == B else jnp.pad(x, ((0, Bp - B), (0, 0)))

    out = pl.pallas_call(
        _noise_linear_kernel,
        grid=(_NCORES, _NSPLIT),
        in_specs=[
            pl.BlockSpec((bt, K), lambda i, j: (i, 0)),
            pl.BlockSpec((K, nh), lambda i, j: (0, j)),
            pl.BlockSpec((K, nh), lambda i, j: (0, j)),
            pl.BlockSpec((1, nh), lambda i, j: (0, j)),
            pl.BlockSpec((1, nh), lambda i, j: (0, j)),
        ],
        out_specs=pl.BlockSpec((bt, nh), lambda i, j: (i, j)),
        out_shape=jax.ShapeDtypeStruct((Bp, N), jnp.float32),
        compiler_params=pltpu.CompilerParams(
            dimension_semantics=("parallel", "arbitrary"),
            vmem_limit_bytes=48 << 20,
        ),
    )(x_p, w_t, noise_w_t, bias2d, noise_b2d)

    return out if Bp == B else out[:B]
